# Initial kernel scaffold; baseline (speedup 1.0000x reference)
import functools

import jax
import jax.numpy as jnp
from jax.experimental import pallas as pl

B, D_IN, T = 16, 512, 2048
NUM_BOOKS, CB_DIM, CB_SIZE = 8, 128, 1024
BETA = 0.25
T_BLK = 512

_HI = jax.lax.Precision.HIGHEST


def _rvq_kernel(z_ref, w_in_ref, b_in_ref, w_out_ref, b_out_ref,
                cb_ref, cbt_ref, mt_ref,
                zq_ref, codes_ref, loss_ref):
    b = pl.program_id(0)
    t = pl.program_id(1)

    @pl.when(jnp.logical_and(b == 0, t == 0))
    def _init():
        loss_ref[0, 0] = jnp.float32(0.0)

    zblk = z_ref[0]
    p = jnp.dot(w_in_ref[...], zblk, preferred_element_type=jnp.float32,
                precision=_HI) + b_in_ref[...]
    qacc = jnp.zeros((CB_DIM, T_BLK), dtype=jnp.float32)
    loss_acc = jnp.float32(0.0)

    row_iota = jax.lax.broadcasted_iota(jnp.int32, (CB_SIZE, T_BLK), 0)

    for i in range(NUM_BOOKS):
        cb = cb_ref[i]
        s = jnp.dot(cb, p, preferred_element_type=jnp.float32, precision=_HI)
        pn = jnp.sum(p * p, axis=0, keepdims=True)
        cbn = jnp.sum(cb * cb, axis=1, keepdims=True)
        d = (pn - 2.0 * s) + cbn
        m = jnp.min(d, axis=0, keepdims=True)
        idx = jnp.min(jnp.where(d == m, row_iota, CB_SIZE), axis=0,
                      keepdims=True)
        codes_ref[0, i:i + 1, :] = idx
        loss_acc = loss_acc + jnp.sum(m)
        onehot = (row_iota == idx).astype(jnp.float32)
        q = jnp.dot(cbt_ref[i], onehot,
                    preferred_element_type=jnp.float32, precision=_HI)
        qacc = qacc + q
        p = p - jnp.dot(mt_ref[...], q,
                        preferred_element_type=jnp.float32, precision=_HI)

    zq_ref[0] = (jnp.dot(w_out_ref[...], qacc,
                         preferred_element_type=jnp.float32, precision=_HI)
                 + jnp.float32(NUM_BOOKS) * b_out_ref[...])
    loss_ref[0, 0] += loss_acc


@functools.partial(jax.jit, static_argnames=())
def kernel(z, W_in, b_in, W_out, b_out, codebooks):
    cbt = jnp.transpose(codebooks, (0, 2, 1))
    mt = jnp.dot(W_in, W_out, precision=_HI)
    b_in_c = b_in.reshape(CB_DIM, 1)
    b_out_c = b_out.reshape(D_IN, 1)

    grid = (B, T // T_BLK)
    zq, codes, loss_raw = pl.pallas_call(
        _rvq_kernel,
        grid=grid,
        in_specs=[
            pl.BlockSpec((1, D_IN, T_BLK), lambda b, t: (b, 0, t)),
            pl.BlockSpec((CB_DIM, D_IN), lambda b, t: (0, 0)),
            pl.BlockSpec((CB_DIM, 1), lambda b, t: (0, 0)),
            pl.BlockSpec((D_IN, CB_DIM), lambda b, t: (0, 0)),
            pl.BlockSpec((D_IN, 1), lambda b, t: (0, 0)),
            pl.BlockSpec((NUM_BOOKS, CB_SIZE, CB_DIM), lambda b, t: (0, 0, 0)),
            pl.BlockSpec((NUM_BOOKS, CB_DIM, CB_SIZE), lambda b, t: (0, 0, 0)),
            pl.BlockSpec((CB_DIM, CB_DIM), lambda b, t: (0, 0)),
        ],
        out_specs=[
            pl.BlockSpec((1, D_IN, T_BLK), lambda b, t: (b, 0, t)),
            pl.BlockSpec((1, NUM_BOOKS, T_BLK), lambda b, t: (b, 0, t)),
            pl.BlockSpec((1, 1), lambda b, t: (0, 0)),
        ],
        out_shape=[
            jax.ShapeDtypeStruct((B, D_IN, T), jnp.float32),
            jax.ShapeDtypeStruct((B, NUM_BOOKS, T), jnp.int32),
            jax.ShapeDtypeStruct((1, 1), jnp.float32),
        ],
    )(z, W_in, b_in_c, W_out, b_out_c, codebooks, cbt, mt)

    scale = (1.0 + BETA) / jnp.float32(B * T * CB_DIM)
    return zq, codes, loss_raw[0, 0] * scale


# speedup vs baseline: 1.9016x; 1.9016x over previous
import functools

import jax
import jax.numpy as jnp
from jax.experimental import pallas as pl

B, D_IN, T = 16, 512, 2048
NUM_BOOKS, CB_DIM, CB_SIZE = 8, 128, 1024
BETA = 0.25
T_BLK = 512


def _dot(a, b):
    return jnp.dot(a, b, preferred_element_type=jnp.float32)


def _rvq_kernel(z_ref, w_in_ref, b_in_ref, w_out_ref, b_out_ref,
                cbt_ref, cbm2_ref, cbn_ref,
                zq_ref, codes_ref, loss_ref):
    b = pl.program_id(0)
    t = pl.program_id(1)

    @pl.when(jnp.logical_and(b == 0, t == 0))
    def _init():
        loss_ref[...] = jnp.zeros((1, 1), dtype=jnp.float32)

    zblk = z_ref[0]
    res = zblk
    loss_acc = jnp.zeros((1, 1), dtype=jnp.float32)
    row_iota = jax.lax.broadcasted_iota(jnp.int32, (CB_SIZE, T_BLK), 0)

    for i in range(NUM_BOOKS):
        p = _dot(w_in_ref[...], res.astype(jnp.bfloat16)) + b_in_ref[...]
        s2 = _dot(cbm2_ref[i], p.astype(jnp.bfloat16))
        pn = jnp.sum(p * p, axis=0, keepdims=True)
        d = (pn + s2) + cbn_ref[:, i:i + 1]
        m = jnp.min(d, axis=0, keepdims=True)
        idx = jnp.min(jnp.where(d == m, row_iota, CB_SIZE), axis=0,
                      keepdims=True)
        codes_ref[0, i:i + 1, :] = idx
        loss_acc = loss_acc + jnp.sum(m, axis=1, keepdims=True)
        lidx = jnp.broadcast_to(idx & (128 - 1), (CB_DIM, T_BLK))
        gidx = jnp.broadcast_to(idx >> 7, (CB_DIM, T_BLK))
        q = jnp.zeros((CB_DIM, T_BLK), dtype=jnp.float32)
        for g in range(CB_SIZE // 128):
            slab = cbt_ref[i][:, g * 128:(g + 1) * 128]
            qg = jnp.take_along_axis(slab, lidx, axis=1)
            q = q + jnp.where(gidx == g, qg, 0.0)
        zq_st = p + (q - p)
        upd = _dot(w_out_ref[...], zq_st.astype(jnp.bfloat16)) + b_out_ref[...]
        res = res - upd

    zq_ref[0] = zblk - res
    loss_ref[...] += loss_acc


@functools.partial(jax.jit, static_argnames=())
def kernel(z, W_in, b_in, W_out, b_out, codebooks):
    w_in_bf = W_in.astype(jnp.bfloat16)
    w_out_bf = W_out.astype(jnp.bfloat16)
    cbm2_bf = (-2.0 * codebooks).astype(jnp.bfloat16)
    cbt = jnp.transpose(codebooks, (0, 2, 1))
    cbn = jnp.transpose(jnp.sum(codebooks * codebooks, axis=2))
    b_in_c = b_in.reshape(CB_DIM, 1)
    b_out_c = b_out.reshape(D_IN, 1)

    grid = (B, T // T_BLK)
    zq, codes, loss_raw = pl.pallas_call(
        _rvq_kernel,
        grid=grid,
        in_specs=[
            pl.BlockSpec((1, D_IN, T_BLK), lambda b, t: (b, 0, t)),
            pl.BlockSpec((CB_DIM, D_IN), lambda b, t: (0, 0)),
            pl.BlockSpec((CB_DIM, 1), lambda b, t: (0, 0)),
            pl.BlockSpec((D_IN, CB_DIM), lambda b, t: (0, 0)),
            pl.BlockSpec((D_IN, 1), lambda b, t: (0, 0)),
            pl.BlockSpec((NUM_BOOKS, CB_DIM, CB_SIZE), lambda b, t: (0, 0, 0)),
            pl.BlockSpec((NUM_BOOKS, CB_SIZE, CB_DIM), lambda b, t: (0, 0, 0)),
            pl.BlockSpec((CB_SIZE, NUM_BOOKS), lambda b, t: (0, 0)),
        ],
        out_specs=[
            pl.BlockSpec((1, D_IN, T_BLK), lambda b, t: (b, 0, t)),
            pl.BlockSpec((1, NUM_BOOKS, T_BLK), lambda b, t: (b, 0, t)),
            pl.BlockSpec((1, 1), lambda b, t: (0, 0)),
        ],
        out_shape=[
            jax.ShapeDtypeStruct((B, D_IN, T), jnp.float32),
            jax.ShapeDtypeStruct((B, NUM_BOOKS, T), jnp.int32),
            jax.ShapeDtypeStruct((1, 1), jnp.float32),
        ],
    )(z, w_in_bf, b_in_c, w_out_bf, b_out_c, cbt, cbm2_bf, cbn)

    scale = (1.0 + BETA) / jnp.float32(B * T * CB_DIM)
    return zq, codes, loss_raw[0, 0] * scale


# speedup vs baseline: 2.8950x; 1.5224x over previous
import functools

import jax
import jax.numpy as jnp
from jax.experimental import pallas as pl

B, D_IN, T = 16, 512, 2048
NUM_BOOKS, CB_DIM, CB_SIZE = 8, 128, 1024
BETA = 0.25
T_BLK = 512
T_SUB = 128
N_SUB = T_BLK // T_SUB


def _dot(a, b):
    return jnp.dot(a, b, preferred_element_type=jnp.float32)


def _gather_q(cbt_i, idx):
    tb = idx.shape[1]
    lidx = jnp.broadcast_to(idx & (128 - 1), (CB_DIM, tb))
    gidx = jnp.broadcast_to(idx >> 7, (CB_DIM, tb))
    qs = [jnp.take_along_axis(cbt_i[:, g * 128:(g + 1) * 128], lidx, axis=1)
          for g in range(CB_SIZE // 128)]
    b0 = (gidx & 1) != 0
    r = [jnp.where(b0, qs[2 * k + 1], qs[2 * k]) for k in range(4)]
    b1 = (gidx & 2) != 0
    s = [jnp.where(b1, r[2 * k + 1], r[2 * k]) for k in range(2)]
    b2 = (gidx & 4) != 0
    return jnp.where(b2, s[1], s[0])


def _stage_lockstep(res, w_in, b_in, w_out, b_out, cbt_i, cbm2_i, cbn_i,
                    row_iota):
    n = len(res)
    p = [_dot(w_in, res[k].astype(jnp.bfloat16)) + b_in for k in range(n)]
    s2 = [_dot(cbm2_i, p[k].astype(jnp.bfloat16)) for k in range(n)]
    pn = [jnp.sum(p[k] * p[k], axis=0, keepdims=True) for k in range(n)]
    d = [(pn[k] + s2[k]) + cbn_i for k in range(n)]
    m = [jnp.min(d[k], axis=0, keepdims=True) for k in range(n)]
    idx = [jnp.min(jnp.where(d[k] == m[k], row_iota, CB_SIZE), axis=0,
                   keepdims=True) for k in range(n)]
    q = [_gather_q(cbt_i, idx[k]) for k in range(n)]
    zq_st = [p[k] + (q[k] - p[k]) for k in range(n)]
    upd = [_dot(w_out, zq_st[k].astype(jnp.bfloat16)) + b_out for k in range(n)]
    new_res = [res[k] - upd[k] for k in range(n)]
    ls = [jnp.sum(m[k], axis=1, keepdims=True) for k in range(n)]
    return new_res, idx, ls


def _rvq_kernel(z_ref, w_in_ref, b_in_ref, w_out_ref, b_out_ref,
                cbt_ref, cbm2_ref, cbn_ref,
                zq_ref, codes_ref, loss_ref):
    b = pl.program_id(0)
    t = pl.program_id(1)

    @pl.when(jnp.logical_and(b == 0, t == 0))
    def _init():
        loss_ref[...] = jnp.zeros((1, 1), dtype=jnp.float32)

    zblk = z_ref[0]
    loss_acc = jnp.zeros((1, 1), dtype=jnp.float32)
    row_iota = jax.lax.broadcasted_iota(jnp.int32, (CB_SIZE, T_SUB), 0)
    w_in = w_in_ref[...]
    b_in = b_in_ref[...]
    w_out = w_out_ref[...]
    b_out = b_out_ref[...]

    res = [zblk[:, k * T_SUB:(k + 1) * T_SUB] for k in range(N_SUB)]
    for i in range(NUM_BOOKS):
        cbt_i = cbt_ref[i]
        cbm2_i = cbm2_ref[i]
        cbn_i = cbn_ref[:, i:i + 1]
        res, idxs, lss = _stage_lockstep(res, w_in, b_in, w_out, b_out,
                                         cbt_i, cbm2_i, cbn_i, row_iota)
        for k in range(N_SUB):
            codes_ref[0, i:i + 1, k * T_SUB:(k + 1) * T_SUB] = idxs[k]
            loss_acc = loss_acc + lss[k]

    zq_ref[0] = zblk - jnp.concatenate(res, axis=1)
    loss_ref[...] += loss_acc


@functools.partial(jax.jit, static_argnames=())
def kernel(z, W_in, b_in, W_out, b_out, codebooks):
    w_in_bf = W_in.astype(jnp.bfloat16)
    w_out_bf = W_out.astype(jnp.bfloat16)
    cbm2_bf = (-2.0 * codebooks).astype(jnp.bfloat16)
    cbt = jnp.transpose(codebooks, (0, 2, 1))
    cbn = jnp.transpose(jnp.sum(codebooks * codebooks, axis=2))
    b_in_c = b_in.reshape(CB_DIM, 1)
    b_out_c = b_out.reshape(D_IN, 1)

    grid = (B, T // T_BLK)
    zq, codes, loss_raw = pl.pallas_call(
        _rvq_kernel,
        grid=grid,
        in_specs=[
            pl.BlockSpec((1, D_IN, T_BLK), lambda b, t: (b, 0, t)),
            pl.BlockSpec((CB_DIM, D_IN), lambda b, t: (0, 0)),
            pl.BlockSpec((CB_DIM, 1), lambda b, t: (0, 0)),
            pl.BlockSpec((D_IN, CB_DIM), lambda b, t: (0, 0)),
            pl.BlockSpec((D_IN, 1), lambda b, t: (0, 0)),
            pl.BlockSpec((NUM_BOOKS, CB_DIM, CB_SIZE), lambda b, t: (0, 0, 0)),
            pl.BlockSpec((NUM_BOOKS, CB_SIZE, CB_DIM), lambda b, t: (0, 0, 0)),
            pl.BlockSpec((CB_SIZE, NUM_BOOKS), lambda b, t: (0, 0)),
        ],
        out_specs=[
            pl.BlockSpec((1, D_IN, T_BLK), lambda b, t: (b, 0, t)),
            pl.BlockSpec((1, NUM_BOOKS, T_BLK), lambda b, t: (b, 0, t)),
            pl.BlockSpec((1, 1), lambda b, t: (0, 0)),
        ],
        out_shape=[
            jax.ShapeDtypeStruct((B, D_IN, T), jnp.float32),
            jax.ShapeDtypeStruct((B, NUM_BOOKS, T), jnp.int32),
            jax.ShapeDtypeStruct((1, 1), jnp.float32),
        ],
    )(z, w_in_bf, b_in_c, w_out_bf, b_out_c, cbt, cbm2_bf, cbn)

    scale = (1.0 + BETA) / jnp.float32(B * T * CB_DIM)
    return zq, codes, loss_raw[0, 0] * scale


# speedup vs baseline: 3.2530x; 1.1237x over previous
import functools

import jax
import jax.numpy as jnp
from jax.experimental import pallas as pl

B, D_IN, T = 16, 512, 2048
NUM_BOOKS, CB_DIM, CB_SIZE = 8, 128, 1024
BETA = 0.25
T_BLK = 1024
T_SUB = 128
N_SUB = T_BLK // T_SUB


def _dot(a, b):
    return jnp.dot(a, b, preferred_element_type=jnp.float32)


def _gather_q(cbt_i, idx):
    tb = idx.shape[1]
    lidx = jnp.broadcast_to(idx & (128 - 1), (CB_DIM, tb))
    gidx = jnp.broadcast_to(idx >> 7, (CB_DIM, tb))
    qs = [jnp.take_along_axis(cbt_i[:, g * 128:(g + 1) * 128], lidx, axis=1)
          for g in range(CB_SIZE // 128)]
    b0 = (gidx & 1) != 0
    r = [jnp.where(b0, qs[2 * k + 1], qs[2 * k]) for k in range(4)]
    b1 = (gidx & 2) != 0
    s = [jnp.where(b1, r[2 * k + 1], r[2 * k]) for k in range(2)]
    b2 = (gidx & 4) != 0
    return jnp.where(b2, s[1], s[0])


def _stage_lockstep(res, w_in, b_in, w_out, b_out, cbt_i, cbm2_i, cbn_i,
                    row_iota):
    n = len(res)
    p = [_dot(w_in, res[k].astype(jnp.bfloat16)) + b_in for k in range(n)]
    s2 = [_dot(cbm2_i, p[k].astype(jnp.bfloat16)) for k in range(n)]
    pn = [jnp.sum(p[k] * p[k], axis=0, keepdims=True) for k in range(n)]
    d = [(pn[k] + s2[k]) + cbn_i for k in range(n)]
    m = [jnp.min(d[k], axis=0, keepdims=True) for k in range(n)]
    idx = [jnp.min(jnp.where(d[k] == m[k], row_iota, CB_SIZE), axis=0,
                   keepdims=True) for k in range(n)]
    q = [_gather_q(cbt_i, idx[k]) for k in range(n)]
    zq_st = [p[k] + (q[k] - p[k]) for k in range(n)]
    upd = [_dot(w_out, zq_st[k].astype(jnp.bfloat16)) + b_out for k in range(n)]
    new_res = [res[k] - upd[k] for k in range(n)]
    ls = [jnp.sum(m[k], axis=1, keepdims=True) for k in range(n)]
    return new_res, idx, ls


def _rvq_kernel(z_ref, w_in_ref, b_in_ref, w_out_ref, b_out_ref,
                cbt_ref, cbm2_ref, cbn_ref,
                zq_ref, codes_ref, loss_ref):
    b = pl.program_id(0)
    t = pl.program_id(1)

    @pl.when(jnp.logical_and(b == 0, t == 0))
    def _init():
        loss_ref[...] = jnp.zeros((1, 1), dtype=jnp.float32)

    zblk = z_ref[0]
    loss_acc = jnp.zeros((1, 1), dtype=jnp.float32)
    row_iota = jax.lax.broadcasted_iota(jnp.int32, (CB_SIZE, T_SUB), 0)
    w_in = w_in_ref[...]
    b_in = b_in_ref[...]
    w_out = w_out_ref[...]
    b_out = b_out_ref[...]

    res = [zblk[:, k * T_SUB:(k + 1) * T_SUB] for k in range(N_SUB)]
    for i in range(NUM_BOOKS):
        cbt_i = cbt_ref[i]
        cbm2_i = cbm2_ref[i]
        cbn_i = cbn_ref[:, i:i + 1]
        res, idxs, lss = _stage_lockstep(res, w_in, b_in, w_out, b_out,
                                         cbt_i, cbm2_i, cbn_i, row_iota)
        for k in range(N_SUB):
            codes_ref[0, i:i + 1, k * T_SUB:(k + 1) * T_SUB] = idxs[k]
            loss_acc = loss_acc + lss[k]

    zq_ref[0] = zblk - jnp.concatenate(res, axis=1)
    loss_ref[...] += loss_acc


@functools.partial(jax.jit, static_argnames=())
def kernel(z, W_in, b_in, W_out, b_out, codebooks):
    w_in_bf = W_in.astype(jnp.bfloat16)
    w_out_bf = W_out.astype(jnp.bfloat16)
    cbm2_bf = (-2.0 * codebooks).astype(jnp.bfloat16)
    cbt = jnp.transpose(codebooks, (0, 2, 1))
    cbn = jnp.transpose(jnp.sum(codebooks * codebooks, axis=2))
    b_in_c = b_in.reshape(CB_DIM, 1)
    b_out_c = b_out.reshape(D_IN, 1)

    grid = (B, T // T_BLK)
    zq, codes, loss_raw = pl.pallas_call(
        _rvq_kernel,
        grid=grid,
        in_specs=[
            pl.BlockSpec((1, D_IN, T_BLK), lambda b, t: (b, 0, t)),
            pl.BlockSpec((CB_DIM, D_IN), lambda b, t: (0, 0)),
            pl.BlockSpec((CB_DIM, 1), lambda b, t: (0, 0)),
            pl.BlockSpec((D_IN, CB_DIM), lambda b, t: (0, 0)),
            pl.BlockSpec((D_IN, 1), lambda b, t: (0, 0)),
            pl.BlockSpec((NUM_BOOKS, CB_DIM, CB_SIZE), lambda b, t: (0, 0, 0)),
            pl.BlockSpec((NUM_BOOKS, CB_SIZE, CB_DIM), lambda b, t: (0, 0, 0)),
            pl.BlockSpec((CB_SIZE, NUM_BOOKS), lambda b, t: (0, 0)),
        ],
        out_specs=[
            pl.BlockSpec((1, D_IN, T_BLK), lambda b, t: (b, 0, t)),
            pl.BlockSpec((1, NUM_BOOKS, T_BLK), lambda b, t: (b, 0, t)),
            pl.BlockSpec((1, 1), lambda b, t: (0, 0)),
        ],
        out_shape=[
            jax.ShapeDtypeStruct((B, D_IN, T), jnp.float32),
            jax.ShapeDtypeStruct((B, NUM_BOOKS, T), jnp.int32),
            jax.ShapeDtypeStruct((1, 1), jnp.float32),
        ],
    )(z, w_in_bf, b_in_c, w_out_bf, b_out_c, cbt, cbm2_bf, cbn)

    scale = (1.0 + BETA) / jnp.float32(B * T * CB_DIM)
    return zq, codes, loss_raw[0, 0] * scale
